# Initial kernel scaffold; baseline (speedup 1.0000x reference)
import jax
import jax.numpy as jnp
from jax.experimental import pallas as pl


def kernel(q, k, v, cu_seqlens, max_seqlen, Wk, Wv, Wg, bg):
    raise NotImplementedError("write your pallas kernel here")


# speedup vs baseline: 1.6903x; 1.6903x over previous
import functools

import jax
import jax.numpy as jnp
import numpy as np
from jax.experimental import pallas as pl
from jax.experimental.pallas import tpu as pltpu

_CSTRIDE = 16
_CBLOCK = 32
_SBLOCK = 64
_NSEL = 8
_WINDOW = 256
_TQ = 128


def _comp_kernel(kcA_ref, kcB_ref, vcA_ref, vcB_ref,
                 wkA_ref, wkB_ref, wvA_ref, wvB_ref, ck_ref, cv_ref):
    dot = lambda a, b: jax.lax.dot_general(
        a, b, (((1,), (0,)), ((), ())), preferred_element_type=jnp.float32,
        precision=jax.lax.Precision.HIGHEST)
    ck_ref[...] = dot(kcA_ref[...], wkA_ref[...]) + dot(kcB_ref[...], wkB_ref[...])
    cv_ref[...] = dot(vcA_ref[...], wvA_ref[...]) + dot(vcB_ref[...], wvB_ref[...])


def _nsa_kernel(q_ref, k_ref, v_ref, ck_ref, cv_ref, wg_ref, bg_ref, o_ref,
                *, tq, s, g, ncpad, nc, nblk, scale):
    qt = pl.program_id(1)
    qs = qt * tq
    f32 = jnp.float32
    i32 = jnp.int32

    dotT = lambda a, b: jax.lax.dot_general(
        a, b, (((1,), (1,)), ((), ())), preferred_element_type=f32)
    dot = lambda a, b: jax.lax.dot_general(
        a, b, (((1,), (0,)), ((), ())), preferred_element_type=f32)
    dotT_hi = lambda a, b: jax.lax.dot_general(
        a, b, (((1,), (1,)), ((), ())), preferred_element_type=f32,
        precision=jax.lax.Precision.HIGHEST)
    dot_hi = lambda a, b: jax.lax.dot_general(
        a, b, (((1,), (0,)), ((), ())), preferred_element_type=f32,
        precision=jax.lax.Precision.HIGHEST)

    rows_t = qs + jax.lax.broadcasted_iota(i32, (tq, 1), 0)

    jc = jax.lax.broadcasted_iota(i32, (tq, ncpad), 1)
    cvalid = (jc * _CSTRIDE + _CBLOCK - 1) <= rows_t
    has_valid = (rows_t >= (_CBLOCK - 1)).astype(f32)

    ck = ck_ref[...]
    cv = cv_ref[...]
    k2 = k_ref[0]
    v2 = v_ref[0]

    cps = []
    for gi in range(g):
        qgi = q_ref[0, gi]
        clog = dotT_hi(qgi, ck) * scale
        clog = jnp.where(cvalid, clog, -1e30)
        cp = jax.nn.softmax(clog, axis=-1) * has_valid
        cps.append(cp)
    score = functools.reduce(lambda a, b: a + b, cps)

    pk = _SBLOCK // _CSTRIDE + 1
    pst = _SBLOCK // _CSTRIDE
    cc = jax.lax.broadcasted_iota(i32, (ncpad, nblk), 0)
    mm = jax.lax.broadcasted_iota(i32, (ncpad, nblk), 1)
    pmask = ((cc >= mm * pst) & (cc <= mm * pst + pk - 1) & (cc < nc)).astype(f32)
    pooled = dot_hi(score, pmask) / jnp.sum(pmask, axis=0, keepdims=True)

    midx = jax.lax.broadcasted_iota(i32, (tq, nblk), 1)
    rank = jnp.zeros((tq, nblk), i32)
    for mp in range(nblk):
        vm = pooled[:, mp:mp + 1]
        rank += (vm > pooled).astype(i32)
        rank += ((vm == pooled) & (mp < midx)).astype(i32)
    selblk = (rank < _NSEL).astype(f32)

    em = jax.lax.broadcasted_iota(i32, (nblk, s), 0)
    ep = jax.lax.broadcasted_iota(i32, (nblk, s), 1)
    emat = (em == ep // _SBLOCK).astype(f32)
    selexp = dot(selblk, emat)

    pcol = jax.lax.broadcasted_iota(i32, (tq, s), 1)
    causal = pcol <= rows_t
    smask = (selexp > 0.5) & causal
    wmask = causal & ((rows_t - pcol) < _WINDOW)

    for gi in range(g):
        qgi = q_ref[0, gi]
        flog = dotT(qgi, k2) * scale
        sp = jax.nn.softmax(jnp.where(smask, flog, -1e30), axis=-1)
        wp = jax.nn.softmax(jnp.where(wmask, flog, -1e30), axis=-1)
        sel_o = dot(sp, v2)
        win_o = dot(wp, v2)
        cmp_o = dot(cps[gi], cv)
        gate = jax.nn.sigmoid(dot(qgi, wg_ref[...]) + bg_ref[...])
        o_ref[0, gi] = (gate[:, 0:1] * sel_o + gate[:, 1:2] * win_o
                        + gate[:, 2:3] * cmp_o)


def kernel(q, k, v, cu_seqlens, max_seqlen, Wk, Wv, Wg, bg):
    bs = cu_seqlens.shape[0] - 1
    total, hq, d = q.shape
    hkv = k.shape[1]
    s = total // bs
    g = hq // hkv
    bh = bs * hkv
    scale = float(1.0 / np.sqrt(d))

    nc = (s - _CBLOCK) // _CSTRIDE + 1
    ncpad = s // _CSTRIDE
    nblk = s // _SBLOCK
    nqt = s // _TQ

    kb = k.reshape(bs, s, hkv, d).transpose(0, 2, 1, 3).reshape(bh, s, d)
    vb = v.reshape(bs, s, hkv, d).transpose(0, 2, 1, 3).reshape(bh, s, d)
    qb = (q.reshape(bs, s, hkv, g, d).transpose(0, 2, 3, 1, 4)
          .reshape(bh, g, s, d))

    kcA = kb.reshape(bh, ncpad, _CSTRIDE * d)
    vcA = vb.reshape(bh, ncpad, _CSTRIDE * d)
    zpad = jnp.zeros((bh, 1, _CSTRIDE * d), jnp.float32)
    kcB = jnp.concatenate([kcA[:, 1:], zpad], axis=1)
    vcB = jnp.concatenate([vcA[:, 1:], zpad], axis=1)
    kcA = kcA.reshape(bh * ncpad, _CSTRIDE * d)
    kcB = kcB.reshape(bh * ncpad, _CSTRIDE * d)
    vcA = vcA.reshape(bh * ncpad, _CSTRIDE * d)
    vcB = vcB.reshape(bh * ncpad, _CSTRIDE * d)
    wkA, wkB = Wk[:_CSTRIDE * d], Wk[_CSTRIDE * d:]
    wvA, wvB = Wv[:_CSTRIDE * d], Wv[_CSTRIDE * d:]

    wg_p = jnp.zeros((d, 8), jnp.float32).at[:, :3].set(Wg)
    bg_p = jnp.zeros((1, 8), jnp.float32).at[0, :3].set(bg)

    ck, cv = pl.pallas_call(
        _comp_kernel,
        grid=(bh,),
        in_specs=[
            pl.BlockSpec((ncpad, _CSTRIDE * d), lambda i: (i, 0)),
            pl.BlockSpec((ncpad, _CSTRIDE * d), lambda i: (i, 0)),
            pl.BlockSpec((ncpad, _CSTRIDE * d), lambda i: (i, 0)),
            pl.BlockSpec((ncpad, _CSTRIDE * d), lambda i: (i, 0)),
            pl.BlockSpec((_CSTRIDE * d, d), lambda i: (0, 0)),
            pl.BlockSpec((_CSTRIDE * d, d), lambda i: (0, 0)),
            pl.BlockSpec((_CSTRIDE * d, d), lambda i: (0, 0)),
            pl.BlockSpec((_CSTRIDE * d, d), lambda i: (0, 0)),
        ],
        out_specs=[
            pl.BlockSpec((ncpad, d), lambda i: (i, 0)),
            pl.BlockSpec((ncpad, d), lambda i: (i, 0)),
        ],
        out_shape=[
            jax.ShapeDtypeStruct((bh * ncpad, d), jnp.float32),
            jax.ShapeDtypeStruct((bh * ncpad, d), jnp.float32),
        ],
    )(kcA, kcB, vcA, vcB, wkA, wkB, wvA, wvB)

    body = functools.partial(_nsa_kernel, tq=_TQ, s=s, g=g, ncpad=ncpad,
                             nc=nc, nblk=nblk, scale=scale)
    o = pl.pallas_call(
        body,
        grid=(bh, nqt),
        in_specs=[
            pl.BlockSpec((1, g, _TQ, d), lambda i, j: (i, 0, j, 0)),
            pl.BlockSpec((1, s, d), lambda i, j: (i, 0, 0)),
            pl.BlockSpec((1, s, d), lambda i, j: (i, 0, 0)),
            pl.BlockSpec((ncpad, d), lambda i, j: (i, 0)),
            pl.BlockSpec((ncpad, d), lambda i, j: (i, 0)),
            pl.BlockSpec((d, 8), lambda i, j: (0, 0)),
            pl.BlockSpec((1, 8), lambda i, j: (0, 0)),
        ],
        out_specs=pl.BlockSpec((1, g, _TQ, d), lambda i, j: (i, 0, j, 0)),
        out_shape=jax.ShapeDtypeStruct((bh, g, s, d), jnp.float32),
        compiler_params=pltpu.CompilerParams(
            dimension_semantics=("parallel", "parallel")),
    )(qb, kb, vb, ck, cv, wg_p, bg_p)

    return (o.reshape(bs, hkv, g, s, d).transpose(0, 3, 1, 2, 4)
            .reshape(total, hq, d))
